# baseline (speedup 1.0000x reference)
import jax
import jax.numpy as jnp
from jax import lax
from jax.experimental import pallas as pl
from jax.experimental.pallas import tpu as pltpu
from jax.experimental.pallas import tpu_sc as plsc

VOCAB = 150
EMBED = 32
BATCH = 16384
HIST = 5
PADV = 152

_NC, _NS, _L = 2, 16, 16
_NW = _NC * _NS
_BW = BATCH // _NW
_CH = _L
_NCH = _BW // _CH


_NH = 2 * HIST
_CW = _NH * _CH


def _sc_body(i_hbm, t_hbm, out_hbm, t_v, acc_v, i_v):
    wid = lax.axis_index("s") * _NC + lax.axis_index("c")
    base = wid * _BW
    pltpu.sync_copy(t_hbm, t_v)
    pltpu.sync_copy(i_hbm.at[pl.ds(wid * _NH * _BW, _NH * _BW)], i_v)

    def group(g, carry2):
        goff = g * _CW
        iv = [i_v[pl.ds(goff + h * _L, _L)] * EMBED for h in range(_NH)]
        for k in range(_L):
            row = g * _L + k
            a0 = iv[0][k]
            acc0 = t_v[pl.ds(a0, _L)]
            acc1 = t_v[pl.ds(a0 + _L, _L)]
            for h in range(1, HIST):
                a = iv[h][k]
                acc0 = acc0 + t_v[pl.ds(a, _L)]
                acc1 = acc1 + t_v[pl.ds(a + _L, _L)]
            b0 = iv[HIST][k]
            acc2 = t_v[pl.ds(b0, _L)]
            acc3 = t_v[pl.ds(b0 + _L, _L)]
            for h in range(HIST + 1, _NH):
                b = iv[h][k]
                acc2 = acc2 + t_v[pl.ds(b, _L)]
                acc3 = acc3 + t_v[pl.ds(b + _L, _L)]
            acc_v[row, pl.ds(0, _L)] = acc0
            acc_v[row, pl.ds(_L, _L)] = acc1
            acc_v[row, pl.ds(2 * _L, _L)] = acc2
            acc_v[row, pl.ds(3 * _L, _L)] = acc3
        return carry2

    lax.fori_loop(0, _NCH, group, 0, unroll=False)
    pltpu.sync_copy(acc_v, out_hbm.at[pl.ds(base, _BW), :])


def _sc_gather(idx_flat, table):
    mesh = plsc.VectorSubcoreMesh(core_axis_name="c", subcore_axis_name="s")
    return pl.kernel(
        _sc_body,
        out_type=jax.ShapeDtypeStruct((BATCH, 2 * EMBED), jnp.float32),
        mesh=mesh,
        compiler_params=pltpu.CompilerParams(needs_layout_passes=False),
        scratch_types=[
            pltpu.VMEM((2 * PADV * EMBED,), jnp.float32),
            pltpu.VMEM((_BW, 2 * EMBED), jnp.float32),
            pltpu.VMEM((_NH * _BW,), jnp.int32),
        ],
    )(idx_flat, table)


_BN = 4096


def _mlp_body(x_ref, w1_ref, b1_ref, w2_ref, b2_ref, w3_ref, b3_ref, out_ref):
    h1 = jnp.maximum(
        jnp.dot(x_ref[...], w1_ref[...], preferred_element_type=jnp.float32)
        + b1_ref[...],
        0.0,
    )
    h2 = jnp.maximum(
        jnp.dot(h1, w2_ref[...], preferred_element_type=jnp.float32)
        + b2_ref[...],
        0.0,
    )
    out_ref[...] = jnp.maximum(
        jnp.dot(h2, w3_ref[...], preferred_element_type=jnp.float32)
        + b3_ref[...],
        0.0,
    )


def _mlp(x, W1, b1, W2, b2, W3, b3):
    grid = (BATCH // _BN,)
    return pl.pallas_call(
        _mlp_body,
        grid=grid,
        in_specs=[
            pl.BlockSpec((_BN, 2 * EMBED), lambda i: (i, 0)),
            pl.BlockSpec(W1.shape, lambda i: (0, 0)),
            pl.BlockSpec((1, EMBED), lambda i: (0, 0)),
            pl.BlockSpec(W2.shape, lambda i: (0, 0)),
            pl.BlockSpec((1, EMBED // 2), lambda i: (0, 0)),
            pl.BlockSpec(W3.shape, lambda i: (0, 0)),
            pl.BlockSpec((1, 1), lambda i: (0, 0)),
        ],
        out_specs=pl.BlockSpec((_BN, 1), lambda i: (i, 0)),
        out_shape=jax.ShapeDtypeStruct((BATCH, 1), jnp.float32),
    )(
        x, W1, b1.reshape(1, EMBED), W2, b2.reshape(1, EMBED // 2), W3,
        b3.reshape(1, 1),
    )


def kernel(radiant_heros, dire_heros, E_r, E_d, W1, b1, W2, b2, W3, b3):
    pad = ((0, PADV - VOCAB), (0, 0))
    table = jnp.concatenate([jnp.pad(E_r, pad), jnp.pad(E_d, pad)]).reshape(-1)
    idx = jnp.concatenate([radiant_heros, dire_heros + PADV], axis=1)
    idx = idx.reshape(_NW, _NCH, _CH, _NH).transpose(0, 1, 3, 2).reshape(-1)
    x = _sc_gather(idx, table)
    return _mlp(x, W1, b1, W2, b2, W3, b3)


# speedup vs baseline: 1.0429x; 1.0429x over previous
import jax
import jax.numpy as jnp
from jax import lax
from jax.experimental import pallas as pl
from jax.experimental.pallas import tpu as pltpu
from jax.experimental.pallas import tpu_sc as plsc

VOCAB = 150
EMBED = 32
BATCH = 16384
HIST = 5
PADV = 152
_TW = EMBED // 2 + 1

_NC, _NS, _L = 2, 16, 16
_NW = _NC * _NS
_BW = BATCH // _NW
_NG = _BW // _L


def _tree_sum(vals):
    while len(vals) > 1:
        vals = [a + b for a, b in zip(vals[::2], vals[1::2])] + (
            [vals[-1]] if len(vals) % 2 else []
        )
    return vals[0]


def _sc_body(r_hbm, d_hbm, t_hbm, out_hbm, r_v, d_v, t_v, acc_v):
    wid = lax.axis_index("s") * _NC + lax.axis_index("c")
    base = wid * _BW
    pltpu.sync_copy(t_hbm, t_v)
    pltpu.sync_copy(r_hbm.at[pl.ds(base * HIST, _BW * HIST)], r_v)
    pltpu.sync_copy(d_hbm.at[pl.ds(base * HIST, _BW * HIST)], d_v)

    lane5 = lax.iota(jnp.int32, _L) * HIST

    def group(g, carry):
        ra = [
            plsc.load_gather(r_v, [lane5 + (g * _L * HIST + h)]) * _TW
            for h in range(HIST)
        ]
        da = [
            plsc.load_gather(d_v, [lane5 + (g * _L * HIST + h)]) * _TW
            + PADV * _TW
            for h in range(HIST)
        ]
        for jp in range(EMBED // 2):
            rg = [
                plsc.unpack(
                    plsc.bitcast(
                        plsc.load_gather(t_v, [a + jp]), jnp.bfloat16
                    ),
                    format=plsc.PackFormat.INTERLEAVED,
                )
                for a in ra
            ]
            dg = [
                plsc.unpack(
                    plsc.bitcast(
                        plsc.load_gather(t_v, [a + jp]), jnp.bfloat16
                    ),
                    format=plsc.PackFormat.INTERLEAVED,
                )
                for a in da
            ]
            acc_v[2 * jp, pl.ds(g * _L, _L)] = _tree_sum([v[0] for v in rg])
            acc_v[2 * jp + 1, pl.ds(g * _L, _L)] = _tree_sum([v[1] for v in rg])
            acc_v[EMBED + 2 * jp, pl.ds(g * _L, _L)] = _tree_sum(
                [v[0] for v in dg]
            )
            acc_v[EMBED + 2 * jp + 1, pl.ds(g * _L, _L)] = _tree_sum(
                [v[1] for v in dg]
            )
        return carry

    lax.fori_loop(0, _NG, group, 0, unroll=False)
    pltpu.sync_copy(acc_v, out_hbm.at[:, pl.ds(base, _BW)])


def _sc_gather(r_flat, d_flat, table_packed):
    mesh = plsc.VectorSubcoreMesh(core_axis_name="c", subcore_axis_name="s")
    return pl.kernel(
        _sc_body,
        out_type=jax.ShapeDtypeStruct((2 * EMBED, BATCH), jnp.float32),
        mesh=mesh,
        compiler_params=pltpu.CompilerParams(needs_layout_passes=False),
        scratch_types=[
            pltpu.VMEM((_BW * HIST,), jnp.int32),
            pltpu.VMEM((_BW * HIST,), jnp.int32),
            pltpu.VMEM((2 * PADV * _TW,), jnp.int32),
            pltpu.VMEM((2 * EMBED, _BW), jnp.float32),
        ],
    )(r_flat, d_flat, table_packed)


_BN = 4096


def _mlp_body(x_ref, w1_ref, b1_ref, w2_ref, b2_ref, w3_ref, b3_ref, out_ref):
    h1 = lax.dot_general(
        w1_ref[...], x_ref[...], (((0,), (0,)), ((), ())),
        preferred_element_type=jnp.float32,
    )
    h1 = jnp.maximum(h1 + b1_ref[...], 0.0)
    h2 = lax.dot_general(
        w2_ref[...], h1, (((0,), (0,)), ((), ())),
        preferred_element_type=jnp.float32,
    )
    h2 = jnp.maximum(h2 + b2_ref[...], 0.0)
    h3 = lax.dot_general(
        w3_ref[...], h2, (((0,), (0,)), ((), ())),
        preferred_element_type=jnp.float32,
    )
    out_ref[...] = jnp.maximum(h3 + b3_ref[...], 0.0)


def _mlp(x, W1, b1, W2, b2, W3, b3):
    grid = (BATCH // _BN,)
    return pl.pallas_call(
        _mlp_body,
        grid=grid,
        in_specs=[
            pl.BlockSpec((2 * EMBED, _BN), lambda i: (0, i)),
            pl.BlockSpec(W1.shape, lambda i: (0, 0)),
            pl.BlockSpec((EMBED, 1), lambda i: (0, 0)),
            pl.BlockSpec(W2.shape, lambda i: (0, 0)),
            pl.BlockSpec((EMBED // 2, 1), lambda i: (0, 0)),
            pl.BlockSpec(W3.shape, lambda i: (0, 0)),
            pl.BlockSpec((1, 1), lambda i: (0, 0)),
        ],
        out_specs=pl.BlockSpec((1, _BN), lambda i: (0, i)),
        out_shape=jax.ShapeDtypeStruct((1, BATCH), jnp.float32),
    )(
        x, W1, b1.reshape(EMBED, 1), W2, b2.reshape(EMBED // 2, 1), W3,
        b3.reshape(1, 1),
    )


def kernel(radiant_heros, dire_heros, E_r, E_d, W1, b1, W2, b2, W3, b3):
    pad = ((0, PADV - VOCAB), (0, 0))
    tb = jnp.concatenate([jnp.pad(E_r, pad), jnp.pad(E_d, pad)])
    tb = jnp.pad(tb.astype(jnp.bfloat16), ((0, 0), (0, 2)))
    t_i32 = lax.bitcast_convert_type(tb.reshape(2 * PADV * _TW, 2), jnp.int32)
    x = _sc_gather(radiant_heros.reshape(-1), dire_heros.reshape(-1), t_i32)
    out = _mlp(x, W1, b1, W2, b2, W3, b3)
    return out.reshape(BATCH, 1)
